# baseline (speedup 1.0000x reference)
import functools

import jax
import jax.numpy as jnp
from jax import lax
from jax.experimental import pallas as pl
from jax.experimental.pallas import tpu as pltpu
from jax.experimental.pallas import tpu_sc as plsc

NUM_CORES = 2
NUM_SUBCORES = 16
NW = NUM_CORES * NUM_SUBCORES

SUB = 128
NSUB = 4
CHUNK = SUB * NSUB


def _emb_body(n_chunks, d, idx_hbm, table_hbm, out_hbm,
              idx_v, rows0, rows1, gsem0, gsem1, osem0, osem1):
  wid = lax.axis_index("s") * NUM_CORES + lax.axis_index("c")
  rows_per_w = n_chunks * CHUNK
  base_row = wid * rows_per_w
  base_idx = wid * (rows_per_w // SUB)

  pltpu.sync_copy(idx_hbm.at[pl.ds(base_idx, rows_per_w // SUB)], idx_v)

  def fire_gathers(c, rows_v, gsem):
    cps = []
    for j in range(NSUB):
      cps.append(pltpu.async_copy(
          table_hbm.at[idx_v.at[c * NSUB + j]],
          rows_v.at[pl.ds(j * SUB, SUB)],
          gsem))
    return cps

  def drain(cps):
    for cp in cps:
      cp.wait()

  def fire_out(c, rows_v, osem):
    return pltpu.async_copy(
        rows_v, out_hbm.at[pl.ds(base_row + c * CHUNK, CHUNK)], osem)

  drain(fire_gathers(0, rows0, gsem0))
  fire_out(0, rows0, osem0)
  drain(fire_gathers(1, rows1, gsem1))
  fire_out(1, rows1, osem1)

  def body(i, carry):
    c = 2 * i
    pltpu.make_async_copy(
        rows0, out_hbm.at[pl.ds(base_row + (c - 2) * CHUNK, CHUNK)],
        osem0).wait()
    drain(fire_gathers(c, rows0, gsem0))
    fire_out(c, rows0, osem0)
    pltpu.make_async_copy(
        rows1, out_hbm.at[pl.ds(base_row + (c - 1) * CHUNK, CHUNK)],
        osem1).wait()
    drain(fire_gathers(c + 1, rows1, gsem1))
    fire_out(c + 1, rows1, osem1)
    return carry

  lax.fori_loop(1, n_chunks // 2, body, 0)

  pltpu.make_async_copy(
      rows0, out_hbm.at[pl.ds(base_row + (n_chunks - 2) * CHUNK, CHUNK)],
      osem0).wait()
  pltpu.make_async_copy(
      rows1, out_hbm.at[pl.ds(base_row + (n_chunks - 1) * CHUNK, CHUNK)],
      osem1).wait()


@jax.jit
def kernel(input_ids, table):
  batch, seq = input_ids.shape
  n_rows, d = table.shape
  b_total = batch * seq
  assert b_total % (NW * CHUNK) == 0
  n_chunks = b_total // (NW * CHUNK)

  idx_flat = input_ids.reshape(b_total // SUB, SUB).astype(jnp.int32)

  mesh = plsc.VectorSubcoreMesh(core_axis_name="c", subcore_axis_name="s")
  run = pl.kernel(
      functools.partial(_emb_body, n_chunks, d),
      out_type=jax.ShapeDtypeStruct((b_total, d), jnp.float32),
      mesh=mesh,
      compiler_params=pltpu.CompilerParams(use_tc_tiling_on_sc=False),
      scratch_types=[
          pltpu.VMEM((b_total // (NW * SUB), SUB), jnp.int32),
          pltpu.VMEM((CHUNK, d), jnp.float32),
          pltpu.VMEM((CHUNK, d), jnp.float32),
          pltpu.SemaphoreType.DMA,
          pltpu.SemaphoreType.DMA,
          pltpu.SemaphoreType.DMA,
          pltpu.SemaphoreType.DMA,
      ],
  )
  out = run(idx_flat, table)
  return out.reshape(batch, seq, d)
